# Initial kernel scaffold; baseline (speedup 1.0000x reference)
import jax
import jax.numpy as jnp
from jax.experimental import pallas as pl


def kernel(inp, tar):
    raise NotImplementedError("write your pallas kernel here")


# speedup vs baseline: 1.2423x; 1.2423x over previous
import jax
import jax.numpy as jnp
from jax import lax
from jax.experimental import pallas as pl
from jax.experimental.pallas import tpu as pltpu
from jax.experimental.pallas import tpu_sc as plsc

_P = 3
_D = 27
_DP = 32
_B = 4
_NQ = 256
_G = 1468
_GP = 1472
_NGRP = _GP // 16
_NW = 32
_QPW = (_B * _NQ) // _NW


def _cubic(t):
    a = -0.75
    at = jnp.abs(t)
    w1 = ((a + 2.0) * at - (a + 3.0)) * at * at + 1.0
    w2 = (((at - 5.0) * at + 8.0) * at - 4.0) * a
    return jnp.where(at <= 1.0, w1, jnp.where(at < 2.0, w2, 0.0))


def _resize1d(x, out_size, axis):
    in_size = x.shape[axis]
    o = jnp.arange(out_size, dtype=jnp.float32)
    src = o * ((in_size - 1) / (out_size - 1))
    i0 = jnp.floor(src).astype(jnp.int32)
    ts = src - i0.astype(jnp.float32)
    offs = jnp.arange(-1, 3)
    idx = jnp.clip(i0[:, None] + offs[None, :], 0, in_size - 1)
    w = _cubic(ts[:, None] - offs[None, :].astype(jnp.float32))
    xm = jnp.moveaxis(x, axis, -1)
    g = xm[..., idx]
    res = jnp.sum(g * w, axis=-1)
    return jnp.moveaxis(res, -1, axis)


def _resize(x, scale):
    _, _, h, w = x.shape
    x = _resize1d(x, int(round(h * scale)), 2)
    x = _resize1d(x, int(round(w * scale)), 3)
    return x


def _unfold(x, p=_P):
    b, c, h, w = x.shape
    nh, nw = h // p, w // p
    x = x.reshape(b, c, nh, p, nw, p)
    x = jnp.transpose(x, (0, 2, 4, 1, 3, 5))
    return x.reshape(b, nh * nw, c, p, p)


def _database(tar, p=_P):
    x2 = _resize(tar, 0.5)
    x4 = _resize(tar, 0.25)
    patches = []
    for i in range(1, p):
        for j in range(1, p):
            for x in (tar, x2, x4):
                if p < min(x.shape[3], x.shape[2]):
                    patches.append(_unfold(x[:, :, i:-(p - i), j:-(p - j)]))
    patches.append(_unfold(tar))
    patches.append(_unfold(x2))
    patches.append(_unfold(x4))
    return jnp.concatenate(patches, axis=1)


def _reassemble(t, h, w, p=_P):
    b, _, c, ph, pw = t.shape
    nh, nw = h // p, w // p
    t = t.reshape(b, nh, nw, c, ph, pw)
    t = jnp.transpose(t, (0, 3, 1, 4, 2, 5))
    return t.reshape(b, c, h, w)


def _sc_nn_kernel(gc_hbm, gr_hbm, t_hbm, i_hbm,
                  sel_hbm, loss_hbm,
                  gc_v, gr_v, t_v, i_v, sel_v, loss_v):
    nc = 2
    wid = lax.axis_index("s") * nc + lax.axis_index("c")
    b = wid // (_NQ // _QPW)
    qbase = (wid % (_NQ // _QPW)) * _QPW

    pltpu.sync_copy(gc_hbm.at[b], gc_v)
    pltpu.sync_copy(gr_hbm.at[b], gr_v)
    pltpu.sync_copy(t_hbm.at[b, pl.ds(qbase, _QPW), :], t_v)
    pltpu.sync_copy(i_hbm.at[b, pl.ds(qbase, _QPW), :], i_v)

    lanes = lax.iota(jnp.int32, 16)

    def q_body(qi, lossacc):
        t_a = t_v[qi, pl.ds(0, 16)]
        t_b = t_v[qi, pl.ds(16, 16)]
        i_a = i_v[qi, pl.ds(0, 16)]
        i_b = i_v[qi, pl.ds(16, 16)]
        m_a = (t_a + i_a) * 0.5
        m_b = (t_b + i_b) * 0.5
        msp = []
        for d in range(_D):
            ms = m_a[d] if d < 16 else m_b[d - 16]
            msp.append(lax.broadcast(ms, (16,)))

        def g_body(gi, carry):
            mv, mg = carry
            accs = [jnp.zeros((16,), jnp.float32) for _ in range(4)]
            base = gi * 16
            for d in range(_D):
                gv = gc_v[d, pl.ds(base, 16)]
                df = msp[d] - gv
                accs[d % 4] = accs[d % 4] + df * df
            score = (accs[0] + accs[1]) + (accs[2] + accs[3])
            better = score < mv
            mv = jnp.where(better, score, mv)
            mg = jnp.where(better, lax.broadcast(gi, (16,)), mg)
            return mv, mg

        mv0 = jnp.full((16,), 3.0e38, jnp.float32)
        mg0 = jnp.zeros((16,), jnp.int32)
        mv, mg = lax.fori_loop(0, _NGRP, g_body, (mv0, mg0))

        gmin = jnp.min(mv)
        cand = mg * 16 + lanes
        masked = jnp.where(mv == gmin, cand, jnp.int32(2**30))
        cstar = jnp.min(masked)

        sel_a = gr_v[cstar, pl.ds(0, 16)]
        sel_b = gr_v[cstar, pl.ds(16, 16)]
        sel_v[qi, pl.ds(0, 16)] = sel_a
        sel_v[qi, pl.ds(16, 16)] = sel_b
        return lossacc + jnp.abs(i_a - sel_a) + jnp.abs(i_b - sel_b)

    lossacc = lax.fori_loop(0, _QPW, q_body, jnp.zeros((16,), jnp.float32))
    loss_v[...] = lossacc
    pltpu.sync_copy(sel_v, sel_hbm.at[b, pl.ds(qbase, _QPW), :])
    pltpu.sync_copy(loss_v, loss_hbm.at[wid])


def kernel(inp, tar):
    h, w = tar.shape[2], tar.shape[3]
    tp = _unfold(tar).reshape(_B, _NQ, _D)
    ip = _unfold(inp).reshape(_B, _NQ, _D)
    g = _database(tar).reshape(_B, _G, _D)

    gr = jnp.concatenate(
        [g, jnp.full((_B, _GP - _G, _D), 1e9, jnp.float32)], axis=1)
    gr = jnp.pad(gr, ((0, 0), (0, 0), (0, _DP - _D)))
    gc = jnp.transpose(gr, (0, 2, 1))
    tr = jnp.pad(tp, ((0, 0), (0, 0), (0, _DP - _D)))
    ir = jnp.pad(ip, ((0, 0), (0, 0), (0, _DP - _D)))

    mesh = plsc.VectorSubcoreMesh(core_axis_name="c", subcore_axis_name="s")
    sel, lossp = pl.kernel(
        _sc_nn_kernel,
        mesh=mesh,
        compiler_params=pltpu.CompilerParams(
            needs_layout_passes=False, use_tc_tiling_on_sc=False),
        out_type=[
            jax.ShapeDtypeStruct((_B, _NQ, _DP), jnp.float32),
            jax.ShapeDtypeStruct((_NW, 16), jnp.float32),
        ],
        scratch_types=[
            pltpu.VMEM((_DP, _GP), jnp.float32),
            pltpu.VMEM((_GP, _DP), jnp.float32),
            pltpu.VMEM((_QPW, _DP), jnp.float32),
            pltpu.VMEM((_QPW, _DP), jnp.float32),
            pltpu.VMEM((_QPW, _DP), jnp.float32),
            pltpu.VMEM((16,), jnp.float32),
        ],
    )(gc, gr, tr, ir)

    selected = sel[:, :, :_D].reshape(_B, _NQ, 3, _P, _P)
    sel_img = _reassemble(selected, h, w)
    loss = lossp.sum() / (_B * _NQ * _D)
    return loss, sel_img
